# baseline (device time: 778043 ns/iter reference)
import jax
import jax.numpy as jnp
from jax import lax
from jax.experimental import pallas as pl
from jax.experimental.pallas import tpu as pltpu

N_DEV = 16
FT = 1024
HALF = 512


def kernel(x, W1, W2):
    m, k = x.shape
    f = W1.shape[1]
    nt = f // FT

    def body(x_ref, w1_ref, w2_ref, out_ref,
             xbr, xbl, abr, abl, w1b, w2b,
             xr_send, xr_recv, xl_send, xl_recv,
             ar_send, ar_recv, al_send, al_recv,
             w1sems, w2sems,
             xr_cred, xl_cred, ar_cred, al_cred):
        my = lax.axis_index("i")
        left = lax.rem(my + (N_DEV - 1), N_DEV)
        right = lax.rem(my + 1, N_DEV)

        barrier_sem = pltpu.get_barrier_semaphore()
        for nbr in (left, right):
            pl.semaphore_signal(
                barrier_sem, inc=1,
                device_id=(nbr,), device_id_type=pl.DeviceIdType.MESH,
            )
        pl.semaphore_wait(barrier_sem, 2)

        def x_rdma(i, buf, send, recv, dev):
            return pltpu.make_async_remote_copy(
                src_ref=buf.at[lax.rem(i, 2)],
                dst_ref=buf.at[lax.rem(i + 1, 2)],
                send_sem=send.at[lax.rem(i, 2)],
                recv_sem=recv.at[lax.rem(i + 1, 2)],
                device_id=(dev,), device_id_type=pl.DeviceIdType.MESH,
            )

        def a_rdma(i, buf, send, recv, dev):
            return pltpu.make_async_remote_copy(
                src_ref=buf.at[lax.rem(i + 1, 2)],
                dst_ref=buf.at[lax.rem(i, 2)],
                send_sem=send.at[lax.rem(i + 1, 2)],
                recv_sem=recv.at[lax.rem(i, 2)],
                device_id=(dev,), device_id_type=pl.DeviceIdType.MESH,
            )

        def xr(i):
            return x_rdma(i, xbr, xr_send, xr_recv, right)

        def xl(i):
            return x_rdma(i, xbl, xl_send, xl_recv, left)

        def ar(i):
            return a_rdma(i, abr, ar_send, ar_recv, right)

        def al(i):
            return a_rdma(i, abl, al_send, al_recv, left)

        def w_copies(t, slot):
            c1 = pltpu.make_async_copy(
                w1_ref.at[:, pl.ds(t * FT, FT)], w1b.at[slot], w1sems.at[slot])
            c2 = pltpu.make_async_copy(
                w2_ref.at[pl.ds(t * FT, FT), :], w2b.at[slot], w2sems.at[slot])
            return c1, c2

        def compute_both(xa, xb, da, db):
            for c in w_copies(0, 0):
                c.start()

            def ft_body(t, _):
                slot = lax.rem(t, 2)

                @pl.when(t + 1 < nt)
                def _():
                    for c in w_copies(t + 1, lax.rem(t + 1, 2)):
                        c.start()

                for c in w_copies(t, slot):
                    c.wait()
                w1v = w1b[slot].astype(jnp.bfloat16)
                w2v = w2b[slot].astype(jnp.bfloat16)
                for src, dst in ((xa, da), (xb, db)):
                    h = jnp.dot(src[...].astype(jnp.bfloat16), w1v,
                                preferred_element_type=jnp.float32)
                    h = h * jax.nn.sigmoid(h)
                    dst[...] = dst[...] + jnp.dot(
                        h.astype(jnp.bfloat16), w2v,
                        preferred_element_type=jnp.float32)
                return 0

            lax.fori_loop(0, nt, ft_body, 0)

        xbr[0] = x_ref[:HALF, :]
        xbl[0] = x_ref[HALF:, :]
        xr(0).start()
        xl(0).start()
        out_ref[...] = jnp.zeros((m, m), jnp.float32)
        compute_both(x_ref.at[pl.ds(0, HALF), :], x_ref.at[pl.ds(HALF, HALF), :],
                     out_ref.at[pl.ds(0, HALF), :], out_ref.at[pl.ds(HALF, HALF), :])
        abr[1] = jnp.zeros((HALF, m), jnp.float32)
        abl[1] = jnp.zeros((HALF, m), jnp.float32)

        def step(s, _):
            s0 = lax.rem(s, 2)
            s1 = lax.rem(s + 1, 2)

            xr(s).wait()
            xl(s).wait()

            @pl.when((s >= 1) & (s <= 13))
            def _():
                pl.semaphore_signal(xr_cred.at[s0], inc=1, device_id=(left,),
                                    device_id_type=pl.DeviceIdType.MESH)
                pl.semaphore_signal(xl_cred.at[s0], inc=1, device_id=(right,),
                                    device_id_type=pl.DeviceIdType.MESH)

            @pl.when(s <= 13)
            def _():
                @pl.when(s >= 1)
                def _():
                    pl.semaphore_wait(xr_cred.at[s0], 1)
                    pl.semaphore_wait(xl_cred.at[s0], 1)
                xr(s + 1).start()
                xl(s + 1).start()

            @pl.when(s >= 1)
            def _():
                ar(s - 1).wait()
                al(s - 1).wait()

            @pl.when(s >= 2)
            def _():
                pl.semaphore_signal(ar_cred.at[s0], inc=1, device_id=(left,),
                                    device_id_type=pl.DeviceIdType.MESH)
                pl.semaphore_signal(al_cred.at[s0], inc=1, device_id=(right,),
                                    device_id_type=pl.DeviceIdType.MESH)

            compute_both(xbr.at[s1], xbl.at[s1], abr.at[s1], abl.at[s1])

            @pl.when(s >= 2)
            def _():
                pl.semaphore_wait(ar_cred.at[s0], 1)
                pl.semaphore_wait(al_cred.at[s0], 1)
            ar(s).start()
            al(s).start()
            return 0

        lax.fori_loop(0, N_DEV - 1, step, 0)

        ar(N_DEV - 2).wait()
        al(N_DEV - 2).wait()
        out_ref[:HALF, :] = out_ref[:HALF, :] + abr[0]
        out_ref[HALF:, :] = out_ref[HALF:, :] + abl[0]

    return pl.pallas_call(
        body,
        out_shape=jax.ShapeDtypeStruct((m, m), jnp.float32),
        in_specs=[
            pl.BlockSpec(memory_space=pltpu.VMEM),
            pl.BlockSpec(memory_space=pl.ANY),
            pl.BlockSpec(memory_space=pl.ANY),
        ],
        out_specs=pl.BlockSpec(memory_space=pltpu.VMEM),
        scratch_shapes=[
            pltpu.VMEM((2, HALF, k), jnp.float32),
            pltpu.VMEM((2, HALF, k), jnp.float32),
            pltpu.VMEM((2, HALF, m), jnp.float32),
            pltpu.VMEM((2, HALF, m), jnp.float32),
            pltpu.VMEM((2, k, FT), jnp.float32),
            pltpu.VMEM((2, FT, m), jnp.float32),
            pltpu.SemaphoreType.DMA((2,)),
            pltpu.SemaphoreType.DMA((2,)),
            pltpu.SemaphoreType.DMA((2,)),
            pltpu.SemaphoreType.DMA((2,)),
            pltpu.SemaphoreType.DMA((2,)),
            pltpu.SemaphoreType.DMA((2,)),
            pltpu.SemaphoreType.DMA((2,)),
            pltpu.SemaphoreType.DMA((2,)),
            pltpu.SemaphoreType.DMA((2,)),
            pltpu.SemaphoreType.DMA((2,)),
            pltpu.SemaphoreType.REGULAR((2,)),
            pltpu.SemaphoreType.REGULAR((2,)),
            pltpu.SemaphoreType.REGULAR((2,)),
            pltpu.SemaphoreType.REGULAR((2,)),
        ],
        compiler_params=pltpu.CompilerParams(
            collective_id=0,
            vmem_limit_bytes=50 * 1024 * 1024,
        ),
    )(x, W1, W2)


# device time: 722185 ns/iter; 1.0773x vs baseline; 1.0773x over previous
import jax
import jax.numpy as jnp
from jax import lax
from jax.experimental import pallas as pl
from jax.experimental.pallas import tpu as pltpu

N_DEV = 16
FT = 1024
HALF = 512


def kernel(x, W1, W2):
    m, k = x.shape
    f = W1.shape[1]
    nt = f // FT

    def body(x_ref, w1_ref, w2_ref, out_ref,
             xbr, xbl, abr, abl, w1b, w2b,
             xr_send, xr_recv, xl_send, xl_recv,
             ar_send, ar_recv, al_send, al_recv,
             w1sems, w2sems,
             xr_cred, xl_cred, ar_cred, al_cred):
        my = lax.axis_index("i")
        left = lax.rem(my + (N_DEV - 1), N_DEV)
        right = lax.rem(my + 1, N_DEV)

        barrier_sem = pltpu.get_barrier_semaphore()
        for nbr in (left, right):
            pl.semaphore_signal(
                barrier_sem, inc=1,
                device_id=(nbr,), device_id_type=pl.DeviceIdType.MESH,
            )
        pl.semaphore_wait(barrier_sem, 2)

        def x_rdma(i, buf, send, recv, dev):
            return pltpu.make_async_remote_copy(
                src_ref=buf.at[lax.rem(i, 2)],
                dst_ref=buf.at[lax.rem(i + 1, 2)],
                send_sem=send.at[lax.rem(i, 2)],
                recv_sem=recv.at[lax.rem(i + 1, 2)],
                device_id=(dev,), device_id_type=pl.DeviceIdType.MESH,
            )

        def a_rdma(i, buf, send, recv, dev):
            return pltpu.make_async_remote_copy(
                src_ref=buf.at[lax.rem(i + 1, 2)],
                dst_ref=buf.at[lax.rem(i, 2)],
                send_sem=send.at[lax.rem(i + 1, 2)],
                recv_sem=recv.at[lax.rem(i, 2)],
                device_id=(dev,), device_id_type=pl.DeviceIdType.MESH,
            )

        def xr(i):
            return x_rdma(i, xbr, xr_send, xr_recv, right)

        def xl(i):
            return x_rdma(i, xbl, xl_send, xl_recv, left)

        def ar(i):
            return a_rdma(i, abr, ar_send, ar_recv, right)

        def al(i):
            return a_rdma(i, abl, al_send, al_recv, left)

        def w_copies(t, slot):
            c1 = pltpu.make_async_copy(
                w1_ref.at[:, pl.ds(t * FT, FT)], w1b.at[slot], w1sems.at[slot])
            c2 = pltpu.make_async_copy(
                w2_ref.at[pl.ds(t * FT, FT), :], w2b.at[slot], w2sems.at[slot])
            return c1, c2

        def compute_both(xa, xb, da, db, prefetch):

            def ft_body(t, _):
                slot = lax.rem(t, 2)

                @pl.when(t + 1 < nt)
                def _():
                    for c in w_copies(t + 1, lax.rem(t + 1, 2)):
                        c.start()

                if prefetch:
                    @pl.when(t + 1 == nt)
                    def _():
                        for c in w_copies(0, 0):
                            c.start()

                for c in w_copies(t, slot):
                    c.wait()
                for src, dst in ((xa, da), (xb, db)):
                    h = jnp.dot(src[...], w1b[slot],
                                preferred_element_type=jnp.float32)
                    h = h * jax.nn.sigmoid(h)
                    dst[...] = dst[...] + jnp.dot(
                        h, w2b[slot], preferred_element_type=jnp.float32)
                return 0

            lax.fori_loop(0, nt, ft_body, 0)

        xbr[0] = x_ref[:HALF, :]
        xbl[0] = x_ref[HALF:, :]
        xr(0).start()
        xl(0).start()
        for c in w_copies(0, 0):
            c.start()
        out_ref[...] = jnp.zeros((m, m), jnp.float32)
        abr[1] = jnp.zeros((HALF, m), jnp.float32)
        abl[1] = jnp.zeros((HALF, m), jnp.float32)

        def step(s, _):
            s0 = lax.rem(s, 2)
            s1 = lax.rem(s + 1, 2)

            xr(s).wait()
            xl(s).wait()

            @pl.when((s >= 1) & (s <= 13))
            def _():
                pl.semaphore_signal(xr_cred.at[s0], inc=1, device_id=(left,),
                                    device_id_type=pl.DeviceIdType.MESH)
                pl.semaphore_signal(xl_cred.at[s0], inc=1, device_id=(right,),
                                    device_id_type=pl.DeviceIdType.MESH)

            @pl.when(s <= 13)
            def _():
                @pl.when(s >= 1)
                def _():
                    pl.semaphore_wait(xr_cred.at[s0], 1)
                    pl.semaphore_wait(xl_cred.at[s0], 1)
                xr(s + 1).start()
                xl(s + 1).start()

            @pl.when(s >= 1)
            def _():
                ar(s - 1).wait()
                al(s - 1).wait()

            @pl.when(s >= 2)
            def _():
                pl.semaphore_signal(ar_cred.at[s0], inc=1, device_id=(left,),
                                    device_id_type=pl.DeviceIdType.MESH)
                pl.semaphore_signal(al_cred.at[s0], inc=1, device_id=(right,),
                                    device_id_type=pl.DeviceIdType.MESH)

            compute_both(xbr.at[s1], xbl.at[s1], abr.at[s1], abl.at[s1],
                         prefetch=True)

            @pl.when(s >= 2)
            def _():
                pl.semaphore_wait(ar_cred.at[s0], 1)
                pl.semaphore_wait(al_cred.at[s0], 1)
            ar(s).start()
            al(s).start()
            return 0

        lax.fori_loop(0, N_DEV - 1, step, 0)

        compute_both(x_ref.at[pl.ds(0, HALF), :], x_ref.at[pl.ds(HALF, HALF), :],
                     out_ref.at[pl.ds(0, HALF), :], out_ref.at[pl.ds(HALF, HALF), :],
                     prefetch=False)
        ar(N_DEV - 2).wait()
        al(N_DEV - 2).wait()
        out_ref[:HALF, :] = out_ref[:HALF, :] + abr[0]
        out_ref[HALF:, :] = out_ref[HALF:, :] + abl[0]

    return pl.pallas_call(
        body,
        out_shape=jax.ShapeDtypeStruct((m, m), jnp.float32),
        in_specs=[
            pl.BlockSpec(memory_space=pltpu.VMEM),
            pl.BlockSpec(memory_space=pl.ANY),
            pl.BlockSpec(memory_space=pl.ANY),
        ],
        out_specs=pl.BlockSpec(memory_space=pltpu.VMEM),
        scratch_shapes=[
            pltpu.VMEM((2, HALF, k), jnp.float32),
            pltpu.VMEM((2, HALF, k), jnp.float32),
            pltpu.VMEM((2, HALF, m), jnp.float32),
            pltpu.VMEM((2, HALF, m), jnp.float32),
            pltpu.VMEM((2, k, FT), jnp.float32),
            pltpu.VMEM((2, FT, m), jnp.float32),
            pltpu.SemaphoreType.DMA((2,)),
            pltpu.SemaphoreType.DMA((2,)),
            pltpu.SemaphoreType.DMA((2,)),
            pltpu.SemaphoreType.DMA((2,)),
            pltpu.SemaphoreType.DMA((2,)),
            pltpu.SemaphoreType.DMA((2,)),
            pltpu.SemaphoreType.DMA((2,)),
            pltpu.SemaphoreType.DMA((2,)),
            pltpu.SemaphoreType.DMA((2,)),
            pltpu.SemaphoreType.DMA((2,)),
            pltpu.SemaphoreType.REGULAR((2,)),
            pltpu.SemaphoreType.REGULAR((2,)),
            pltpu.SemaphoreType.REGULAR((2,)),
            pltpu.SemaphoreType.REGULAR((2,)),
        ],
        compiler_params=pltpu.CompilerParams(
            collective_id=0,
            vmem_limit_bytes=50 * 1024 * 1024,
        ),
    )(x, W1, W2)
